# Initial kernel scaffold; baseline (speedup 1.0000x reference)
import jax
import jax.numpy as jnp
from jax.experimental import pallas as pl


def kernel(loc_preds, conf_preds, ground_truth, priors):
    raise NotImplementedError("write your pallas kernel here")


# speedup vs baseline: 7.2099x; 7.2099x over previous
import functools

import jax
import jax.numpy as jnp
from jax.experimental import pallas as pl
from jax.experimental.pallas import tpu as pltpu

_B = 32
_P = 8732
_C = 81
_A = 48
_PP = 8960
_S = 1120
_PT = _PP // _S
_THRESHOLD = 0.5
_NEG_POS_RATIO = 3
_VAR0 = 0.1
_VAR1 = 0.2


def _match_kernel(gt_ref, pr_ref, loc_ref, conf_out, loc_out):
    gt = gt_ref[0]
    tx1 = gt[:, 0:1]
    ty1 = gt[:, 1:2]
    tx2 = gt[:, 2:3]
    ty2 = gt[:, 3:4]
    lab = gt[:, 4:5]

    cx = pr_ref[0:1, :]
    cy = pr_ref[1:2, :]
    pw = pr_ref[2:3, :]
    ph = pr_ref[3:4, :]
    px1 = cx - pw * 0.5
    py1 = cy - ph * 0.5
    px2 = cx + pw * 0.5
    py2 = cy + ph * 0.5

    iw = jnp.maximum(jnp.minimum(tx2, px2) - jnp.maximum(tx1, px1), 0.0)
    ih = jnp.maximum(jnp.minimum(ty2, py2) - jnp.maximum(ty1, py1), 0.0)
    inter = iw * ih
    area_a = (tx2 - tx1) * (ty2 - ty1)
    area_b = pw * ph
    ov = inter / (area_a + area_b - inter)

    best_prior_idx = jnp.argmax(ov, axis=1).reshape(_A, 1)
    bto = jnp.max(ov, axis=0, keepdims=True)
    bti = jnp.argmax(ov, axis=0).reshape(1, _PP)

    j_iota = jax.lax.broadcasted_iota(jnp.int32, (_A, _PP), 0)
    p_iota = jax.lax.broadcasted_iota(jnp.int32, (_A, _PP), 1)
    eq = p_iota == best_prior_idx
    forced = jnp.max(eq.astype(jnp.int32), axis=0, keepdims=True) > 0
    jstar = jnp.max(jnp.where(eq, j_iota, -1), axis=0, keepdims=True)
    bti = jnp.where(forced, jstar, bti)
    bto = jnp.where(forced, 2.0, bto)

    oh = (j_iota == bti).astype(jnp.float32)
    mx1 = jnp.sum(oh * tx1, axis=0, keepdims=True)
    my1 = jnp.sum(oh * ty1, axis=0, keepdims=True)
    mx2 = jnp.sum(oh * tx2, axis=0, keepdims=True)
    my2 = jnp.sum(oh * ty2, axis=0, keepdims=True)
    mlab = jnp.sum(oh * lab, axis=0, keepdims=True)

    conf = jnp.where(bto < _THRESHOLD, 0, mlab.astype(jnp.int32) + 1)
    conf_out[0] = conf

    g_cx = ((mx1 + mx2) * 0.5 - cx) / (_VAR0 * pw)
    g_cy = ((my1 + my2) * 0.5 - cy) / (_VAR0 * ph)
    g_w = jnp.log((mx2 - mx1) / pw) / _VAR1
    g_h = jnp.log((my2 - my1) / ph) / _VAR1

    lp = loc_ref[0]
    d0 = lp[0:1, :] - g_cx
    d1 = lp[1:2, :] - g_cy
    d2 = lp[2:3, :] - g_w
    d3 = lp[3:4, :] - g_h

    def sl1(d):
        a = jnp.abs(d)
        return jnp.where(a < 1.0, 0.5 * d * d, a - 0.5)

    posm = (conf > 0).astype(jnp.float32)
    tot = (sl1(d0) + sl1(d1) + sl1(d2) + sl1(d3)) * posm
    loc_out[0] = jnp.sum(tot, keepdims=True)


def _nll_kernel(x_ref, ct_ref, out_ref):
    j = pl.program_id(1)
    x = x_ref[0]
    ct = ct_ref[0]
    m = jnp.max(x, axis=1, keepdims=True)
    s = jnp.sum(jnp.exp(x - m), axis=1, keepdims=True)
    lse = jnp.log(s) + m
    cls_iota = jax.lax.broadcasted_iota(jnp.int32, (_S, _C), 1)
    tgt = jnp.sum(jnp.where(cls_iota == ct, x, 0.0), axis=1, keepdims=True)
    row = jax.lax.broadcasted_iota(jnp.int32, (_S, 1), 0) + j * _S
    out_ref[0] = jnp.where(row < _P, lse - tgt, 0.0)


def _mine_kernel(nll_ref, ct_ref, locv_ref, loc_out, conf_out):
    nll = nll_ref[...]
    ct = ct_ref[...]
    pos = ct > 0
    np_row = jnp.sum(pos.astype(jnp.int32), axis=0, keepdims=True)
    k = jnp.minimum(_NEG_POS_RATIO * np_row, _P - 1)
    v = jnp.where(pos, 0.0, nll)

    def body(_, carry):
        lo, hi = carry
        mid = lo + ((hi - lo + 1) >> 1)
        t = jax.lax.bitcast_convert_type(mid, jnp.float32)
        cnt = jnp.sum((v >= t).astype(jnp.int32), axis=0, keepdims=True)
        ok = cnt >= k
        return jnp.where(ok, mid, lo), jnp.where(ok, hi, mid - 1)

    lo = jnp.zeros((1, _B), jnp.int32)
    hi = jnp.full((1, _B), 0x7F7FFFFF, jnp.int32)
    lo, hi = jax.lax.fori_loop(0, 31, body, (lo, hi))
    t = jax.lax.bitcast_convert_type(lo, jnp.float32)

    neg_sum = jnp.sum(jnp.where(v >= t, v, 0.0), keepdims=True)
    pos_sum = jnp.sum(jnp.where(pos, nll, 0.0), keepdims=True)
    n = jnp.maximum(jnp.sum(np_row, keepdims=True).astype(jnp.float32), 1.0)
    loc_out[...] = jnp.sum(locv_ref[...], keepdims=True) / n
    conf_out[...] = (neg_sum + pos_sum) / n


@jax.jit
def kernel(loc_preds, conf_preds, ground_truth, priors):
    pad = _PP - _P
    pr_pad = jnp.concatenate(
        [
            jnp.full((2, pad), -100.0, jnp.float32),
            jnp.ones((2, pad), jnp.float32),
        ],
        axis=0,
    )
    pr_t = jnp.concatenate([priors.T, pr_pad], axis=1)
    loc_t = jnp.pad(loc_preds.transpose(0, 2, 1), ((0, 0), (0, 0), (0, pad)))

    conf_t, loc_part = pl.pallas_call(
        _match_kernel,
        grid=(_B,),
        in_specs=[
            pl.BlockSpec((1, _A, 5), lambda b: (b, 0, 0)),
            pl.BlockSpec((4, _PP), lambda b: (0, 0)),
            pl.BlockSpec((1, 4, _PP), lambda b: (b, 0, 0)),
        ],
        out_specs=[
            pl.BlockSpec((1, 1, _PP), lambda b: (b, 0, 0)),
            pl.BlockSpec((1, 1, 1), lambda b: (b, 0, 0)),
        ],
        out_shape=[
            jax.ShapeDtypeStruct((_B, 1, _PP), jnp.int32),
            jax.ShapeDtypeStruct((_B, 1, 1), jnp.float32),
        ],
    )(ground_truth, pr_t, loc_t)

    ct_cm = conf_t.reshape(_B, _PP)[..., None]

    nll = pl.pallas_call(
        _nll_kernel,
        grid=(_B, _PT),
        in_specs=[
            pl.BlockSpec((1, _S, _C), lambda b, j: (b, j, 0)),
            pl.BlockSpec((1, _S, 1), lambda b, j: (b, j, 0)),
        ],
        out_specs=pl.BlockSpec((1, _S, 1), lambda b, j: (b, j, 0)),
        out_shape=jax.ShapeDtypeStruct((_B, _PP, 1), jnp.float32),
    )(conf_preds, ct_cm)

    nll_cm = nll.reshape(_B, _PP).T
    ct_t = conf_t.reshape(_B, _PP).T
    locv = loc_part.reshape(1, _B)

    loc_loss, conf_loss = pl.pallas_call(
        _mine_kernel,
        out_specs=[
            pl.BlockSpec((1, 1), lambda: (0, 0)),
            pl.BlockSpec((1, 1), lambda: (0, 0)),
        ],
        out_shape=[
            jax.ShapeDtypeStruct((1, 1), jnp.float32),
            jax.ShapeDtypeStruct((1, 1), jnp.float32),
        ],
    )(nll_cm, ct_t, locv)

    return loc_loss[0, 0], conf_loss[0, 0]


# speedup vs baseline: 7.7781x; 1.0788x over previous
import functools

import jax
import jax.numpy as jnp
from jax.experimental import pallas as pl
from jax.experimental.pallas import tpu as pltpu

_B = 32
_P = 8732
_C = 81
_A = 48
_PP = 8960
_S = 1120
_PT = _PP // _S
_THRESHOLD = 0.5
_NEG_POS_RATIO = 3
_VAR0 = 0.1
_VAR1 = 0.2


def _match_kernel(gt_ref, pr_ref, loc_ref, conf_out, loc_out):
    gt = gt_ref[0]
    tx1 = gt[:, 0:1]
    ty1 = gt[:, 1:2]
    tx2 = gt[:, 2:3]
    ty2 = gt[:, 3:4]
    lab = gt[:, 4:5]

    cx = pr_ref[0:1, :]
    cy = pr_ref[1:2, :]
    pw = pr_ref[2:3, :]
    ph = pr_ref[3:4, :]
    px1 = cx - pw * 0.5
    py1 = cy - ph * 0.5
    px2 = cx + pw * 0.5
    py2 = cy + ph * 0.5

    iw = jnp.maximum(jnp.minimum(tx2, px2) - jnp.maximum(tx1, px1), 0.0)
    ih = jnp.maximum(jnp.minimum(ty2, py2) - jnp.maximum(ty1, py1), 0.0)
    inter = iw * ih
    area_a = (tx2 - tx1) * (ty2 - ty1)
    area_b = pw * ph
    ov = inter / (area_a + area_b - inter)

    best_prior_idx = jnp.argmax(ov, axis=1).reshape(_A, 1)
    bto = jnp.max(ov, axis=0, keepdims=True)
    bti = jnp.argmax(ov, axis=0).reshape(1, _PP)

    j_iota = jax.lax.broadcasted_iota(jnp.int32, (_A, _PP), 0)
    p_iota = jax.lax.broadcasted_iota(jnp.int32, (_A, _PP), 1)
    eq = p_iota == best_prior_idx
    forced = jnp.max(eq.astype(jnp.int32), axis=0, keepdims=True) > 0
    jstar = jnp.max(jnp.where(eq, j_iota, -1), axis=0, keepdims=True)
    bti = jnp.where(forced, jstar, bti)
    bto = jnp.where(forced, 2.0, bto)

    oh = (j_iota == bti).astype(jnp.float32)
    mx1 = jnp.sum(oh * tx1, axis=0, keepdims=True)
    my1 = jnp.sum(oh * ty1, axis=0, keepdims=True)
    mx2 = jnp.sum(oh * tx2, axis=0, keepdims=True)
    my2 = jnp.sum(oh * ty2, axis=0, keepdims=True)
    mlab = jnp.sum(oh * lab, axis=0, keepdims=True)

    conf = jnp.where(bto < _THRESHOLD, 0, mlab.astype(jnp.int32) + 1)
    conf_out[0] = conf

    g_cx = ((mx1 + mx2) * 0.5 - cx) / (_VAR0 * pw)
    g_cy = ((my1 + my2) * 0.5 - cy) / (_VAR0 * ph)
    g_w = jnp.log((mx2 - mx1) / pw) / _VAR1
    g_h = jnp.log((my2 - my1) / ph) / _VAR1

    lp = loc_ref[0]
    d0 = lp[0:1, :] - g_cx
    d1 = lp[1:2, :] - g_cy
    d2 = lp[2:3, :] - g_w
    d3 = lp[3:4, :] - g_h

    def sl1(d):
        a = jnp.abs(d)
        return jnp.where(a < 1.0, 0.5 * d * d, a - 0.5)

    posm = (conf > 0).astype(jnp.float32)
    tot = (sl1(d0) + sl1(d1) + sl1(d2) + sl1(d3)) * posm
    loc_out[0] = jnp.sum(tot, keepdims=True)


def _nll_kernel(x_ref, ct_ref, out_ref):
    j = pl.program_id(1)
    x = x_ref[0]
    ct = ct_ref[0]
    m = jnp.max(x, axis=1, keepdims=True)
    s = jnp.sum(jnp.exp(x - m), axis=1, keepdims=True)
    lse = jnp.log(s) + m
    cls_iota = jax.lax.broadcasted_iota(jnp.int32, (_S, _C), 1)
    tgt = jnp.sum(jnp.where(cls_iota == ct, x, 0.0), axis=1, keepdims=True)
    row = jax.lax.broadcasted_iota(jnp.int32, (_S, 1), 0) + j * _S
    out_ref[0] = jnp.where(row < _P, lse - tgt, 0.0)


def _mine_kernel(nll_ref, ct_ref, locv_ref, loc_out, conf_out):
    nll = nll_ref[...]
    ct = ct_ref[...]
    pos = ct > 0
    np_row = jnp.sum(pos.astype(jnp.int32), axis=1, keepdims=True)
    k = jnp.minimum(_NEG_POS_RATIO * np_row, _P - 1)
    v = jnp.where(pos, 0.0, nll)

    def body(_, carry):
        lo, hi = carry
        mid = lo + ((hi - lo + 1) >> 1)
        t = jax.lax.bitcast_convert_type(mid, jnp.float32)
        cnt = jnp.sum((v >= t).astype(jnp.int32), axis=1, keepdims=True)
        ok = cnt >= k
        return jnp.where(ok, mid, lo), jnp.where(ok, hi, mid - 1)

    lo = jnp.zeros((_B, 1), jnp.int32)
    hi = jnp.full((_B, 1), 0x7F7FFFFF, jnp.int32)
    lo, hi = jax.lax.fori_loop(0, 31, body, (lo, hi))
    t = jax.lax.bitcast_convert_type(lo, jnp.float32)

    neg_sum = jnp.sum(jnp.where(v >= t, v, 0.0), keepdims=True)
    pos_sum = jnp.sum(jnp.where(pos, nll, 0.0), keepdims=True)
    n = jnp.maximum(jnp.sum(np_row, keepdims=True).astype(jnp.float32), 1.0)
    loc_out[...] = jnp.sum(locv_ref[...], keepdims=True) / n
    conf_out[...] = (neg_sum + pos_sum) / n


@jax.jit
def kernel(loc_preds, conf_preds, ground_truth, priors):
    pad = _PP - _P
    pr_pad = jnp.concatenate(
        [
            jnp.full((2, pad), -100.0, jnp.float32),
            jnp.ones((2, pad), jnp.float32),
        ],
        axis=0,
    )
    pr_t = jnp.concatenate([priors.T, pr_pad], axis=1)
    loc_t = jnp.pad(loc_preds.transpose(0, 2, 1), ((0, 0), (0, 0), (0, pad)))

    conf_t, loc_part = pl.pallas_call(
        _match_kernel,
        grid=(_B,),
        in_specs=[
            pl.BlockSpec((1, _A, 5), lambda b: (b, 0, 0)),
            pl.BlockSpec((4, _PP), lambda b: (0, 0)),
            pl.BlockSpec((1, 4, _PP), lambda b: (b, 0, 0)),
        ],
        out_specs=[
            pl.BlockSpec((1, 1, _PP), lambda b: (b, 0, 0)),
            pl.BlockSpec((1, 1, 1), lambda b: (b, 0, 0)),
        ],
        out_shape=[
            jax.ShapeDtypeStruct((_B, 1, _PP), jnp.int32),
            jax.ShapeDtypeStruct((_B, 1, 1), jnp.float32),
        ],
        compiler_params=pltpu.CompilerParams(
            dimension_semantics=("parallel",)
        ),
    )(ground_truth, pr_t, loc_t)

    ct_cm = conf_t.reshape(_B, _PP)[..., None]

    nll = pl.pallas_call(
        _nll_kernel,
        grid=(_B, _PT),
        in_specs=[
            pl.BlockSpec((1, _S, _C), lambda b, j: (b, j, 0)),
            pl.BlockSpec((1, _S, 1), lambda b, j: (b, j, 0)),
        ],
        out_specs=pl.BlockSpec((1, _S, 1), lambda b, j: (b, j, 0)),
        out_shape=jax.ShapeDtypeStruct((_B, _PP, 1), jnp.float32),
        compiler_params=pltpu.CompilerParams(
            dimension_semantics=("parallel", "parallel")
        ),
    )(conf_preds, ct_cm)

    nll_lm = nll.reshape(_B, _PP)
    ct_lm = conf_t.reshape(_B, _PP)
    locv = loc_part.reshape(1, _B)

    loc_loss, conf_loss = pl.pallas_call(
        _mine_kernel,
        out_specs=[
            pl.BlockSpec((1, 1), lambda: (0, 0)),
            pl.BlockSpec((1, 1), lambda: (0, 0)),
        ],
        out_shape=[
            jax.ShapeDtypeStruct((1, 1), jnp.float32),
            jax.ShapeDtypeStruct((1, 1), jnp.float32),
        ],
    )(nll_lm, ct_lm, locv)

    return loc_loss[0, 0], conf_loss[0, 0]
